# Initial kernel scaffold; baseline (speedup 1.0000x reference)
import jax
import jax.numpy as jnp
from jax.experimental import pallas as pl


def kernel(ind_2, px, i1, diff, w_pp):
    raise NotImplementedError("write your pallas kernel here")


# speedup vs baseline: 25.5385x; 25.5385x over previous
import jax
import jax.numpy as jnp
from jax import lax
from jax.experimental import pallas as pl
from jax.experimental.pallas import tpu as pltpu
from jax.experimental.pallas import tpu_sc as plsc

N_CORES = 2
N_SUB = 16
LANES = 16
CHUNK = 128
GLANES = 4


def _sc_pass(ind_i, ind_j, px_t, i1, diff_flat, *, n_pairs, n_atoms, d_feat):
    per_core = n_pairs // N_CORES
    chunks_per_core = per_core // CHUNK
    base_chunks = chunks_per_core // N_SUB
    extra = chunks_per_core - base_chunks * N_SUB
    assert n_atoms == 640 * (N_SUB - 1) + 400

    mesh = plsc.VectorSubcoreMesh(
        core_axis_name="c", subcore_axis_name="s",
        num_cores=N_CORES, num_subcores=N_SUB)

    def body(indi_ref, indj_ref, px_ref, i1_ref, diff_ref,
             ix_ref, praw_ref,
             jidx, iidx, dfb, rows, i1b, outb, acc, sem):
        c = lax.axis_index("c")
        s = lax.axis_index("s")
        my_chunks = base_chunks + jnp.where(s < extra, 1, 0)
        lo = s * 640

        def plane(x, _):
            zero = jnp.zeros((LANES,), jnp.float32)

            def zrow(i, _):
                for k in range(d_feat // LANES):
                    outb[i, pl.ds(k * LANES, LANES)] = zero
                return 0

            lax.fori_loop(0, CHUNK, zrow, 0)
            for j in range(3):
                pltpu.sync_copy(outb, acc.at[pl.ds(lo + j * CHUNK, CHUNK)])

            @pl.when(s < N_SUB - 1)
            def _():
                for j in (3, 4):
                    pltpu.sync_copy(outb, acc.at[pl.ds(lo + j * CHUNK, CHUNK)])

            @pl.when(s == N_SUB - 1)
            def _():
                pltpu.sync_copy(outb.at[pl.ds(0, 16)],
                                acc.at[pl.ds(lo + 3 * CHUNK, 16)])

            plsc.subcore_barrier()

            def process(t, _):
                base = c * per_core + (s + t * N_SUB) * CHUNK
                pltpu.sync_copy(indj_ref.at[pl.ds(base, CHUNK)], jidx)
                pltpu.sync_copy(indi_ref.at[pl.ds(base, CHUNK)], iidx)
                gather = pltpu.async_copy(px_ref.at[x].at[jidx], rows, sem)
                pltpu.sync_copy(
                    diff_ref.at[pl.ds(x * n_pairs + base, CHUNK)],
                    dfb.at[pl.ds(0, CHUNK)])
                pltpu.sync_copy(i1_ref.at[pl.ds(base, CHUNK), :], i1b)
                gather.wait()

                def group(g, _):
                    dvec = dfb[pl.ds(g * GLANES, LANES)]
                    for l in range(GLANES):
                        i = g * GLANES + l
                        dv = lax.broadcast_in_dim(dvec[l], (LANES,), ())
                        for k in range(d_feat // LANES):
                            r = rows[i, pl.ds(k * LANES, LANES)]
                            a = i1b[i, pl.ds(k * LANES, LANES)]
                            outb[i, pl.ds(k * LANES, LANES)] = (r + dv) * a
                    return 0

                lax.fori_loop(0, CHUNK // GLANES, group, 0)

                pltpu.sync_copy(outb, ix_ref.at[x].at[pl.ds(base, CHUNK), :])
                pltpu.sync_copy(outb, acc.at[iidx], add=True)
                return 0

            lax.fori_loop(0, my_chunks, process, 0)
            plsc.subcore_barrier()

            dst = praw_ref.at[x].at[c]
            for j in range(3):
                pltpu.sync_copy(acc.at[pl.ds(lo + j * CHUNK, CHUNK)],
                                dst.at[pl.ds(lo + j * CHUNK, CHUNK), :])

            @pl.when(s < N_SUB - 1)
            def _():
                for j in (3, 4):
                    pltpu.sync_copy(acc.at[pl.ds(lo + j * CHUNK, CHUNK)],
                                    dst.at[pl.ds(lo + j * CHUNK, CHUNK), :])

            @pl.when(s == N_SUB - 1)
            def _():
                pltpu.sync_copy(acc.at[pl.ds(lo + 3 * CHUNK, 16)],
                                dst.at[pl.ds(lo + 3 * CHUNK, 16), :])

            return 0

        lax.fori_loop(0, 3, plane, 0)

    fn = pl.kernel(
        body,
        out_type=[
            jax.ShapeDtypeStruct((3, n_pairs, d_feat), jnp.float32),
            jax.ShapeDtypeStruct((3, N_CORES, n_atoms, d_feat), jnp.float32),
        ],
        mesh=mesh,
        scratch_types=[
            pltpu.VMEM((CHUNK,), jnp.int32),
            pltpu.VMEM((CHUNK,), jnp.int32),
            pltpu.VMEM((CHUNK + LANES,), jnp.float32),
            pltpu.VMEM((CHUNK, d_feat), jnp.float32),
            pltpu.VMEM((CHUNK, d_feat), jnp.float32),
            pltpu.VMEM((CHUNK, d_feat), jnp.float32),
            pltpu.VMEM_SHARED((n_atoms, d_feat), jnp.float32),
            pltpu.SemaphoreType.DMA,
        ],
    )
    return fn(ind_i, ind_j, px_t, i1, diff_flat)


def _tc_finish(praw_parts, w_pp, *, blk=400):
    three, ncores, n_atoms, d_feat = praw_parts.shape
    n_out = w_pp.shape[1]

    def body(parts_ref, w_ref, p_ref, dot_ref):
        xs = parts_ref[...]
        acc = xs[:, 0] + xs[:, 1]
        y = lax.dot_general(acc.reshape(three * blk, d_feat), w_ref[...],
                            (((1,), (0,)), ((), ())),
                            preferred_element_type=jnp.float32,
                            precision=lax.Precision.HIGHEST)
        y3 = y.reshape(three, blk, n_out)
        p_ref[...] = y3
        dot_ref[...] = jnp.sum(y3 * y3, axis=0)

    return pl.pallas_call(
        body,
        grid=(n_atoms // blk,),
        in_specs=[
            pl.BlockSpec((three, ncores, blk, d_feat), lambda i: (0, 0, i, 0)),
            pl.BlockSpec((d_feat, n_out), lambda i: (0, 0)),
        ],
        out_specs=[
            pl.BlockSpec((three, blk, n_out), lambda i: (0, i, 0)),
            pl.BlockSpec((blk, n_out), lambda i: (i, 0)),
        ],
        out_shape=[
            jax.ShapeDtypeStruct((three, n_atoms, n_out), jnp.float32),
            jax.ShapeDtypeStruct((n_atoms, n_out), jnp.float32),
        ],
    )(praw_parts, w_pp)


def kernel(ind_2, px, i1, diff, w_pp):
    n_atoms, three, d_feat = px.shape
    n_pairs = i1.shape[0]

    ind_i = ind_2[:, 0].astype(jnp.int32)
    ind_j = ind_2[:, 1].astype(jnp.int32)
    px_t = jnp.transpose(px, (1, 0, 2))
    diff_flat = jnp.transpose(diff, (1, 0)).reshape(-1)

    ix_t, praw_parts = _sc_pass(
        ind_i, ind_j, px_t, i1, diff_flat,
        n_pairs=n_pairs, n_atoms=n_atoms, d_feat=d_feat)
    p_t, dotted = _tc_finish(praw_parts, w_pp)

    ix = jnp.transpose(ix_t, (1, 0, 2))
    p = jnp.transpose(p_t, (1, 0, 2))
    return (p, ix, dotted)
